# Initial kernel scaffold; baseline (speedup 1.0000x reference)
import jax
import jax.numpy as jnp
from jax.experimental import pallas as pl


def kernel(x, edge_index, Wl1, Wr1, att1, b1, Wl2, Wr2, att2, b2):
    raise NotImplementedError("write your pallas kernel here")


# speedup vs baseline: 19.8432x; 19.8432x over previous
import functools

import numpy as np
import jax
import jax.numpy as jnp
from jax import lax
from jax.experimental import pallas as pl
from jax.experimental.pallas import tpu as pltpu
from jax.experimental.pallas import tpu_sc as plsc

_f32 = jnp.float32
_i32 = jnp.int32

N = 10000
E = 160000
IN = 4
H1, D1 = 8, 120
F1 = H1 * D1
D2 = 4

NC, NS = 2, 16
NW = NC * NS
E_PAD = 163840
EW = E_PAD // NW
EC = 2560
N_PAD = 10240
NR = N_PAD // NS

BE = 2560
BN = 2000


def _mesh():
    return plsc.VectorSubcoreMesh(
        core_axis_name="c", subcore_axis_name="s",
        num_cores=NC, num_subcores=NS)


def _sc_gather(tab_s, tab_d, src, dst):
    out_t = (jax.ShapeDtypeStruct((E_PAD, 16), _f32),
             jax.ShapeDtypeStruct((E_PAD, 16), _f32))

    @functools.partial(
        pl.kernel, out_type=out_t, mesh=_mesh(),
        compiler_params=pltpu.CompilerParams(use_tc_tiling_on_sc=False),
        scratch_types=[pltpu.VMEM((EC,), _i32),
                       pltpu.VMEM((EC, 16), _f32),
                       pltpu.SemaphoreType.DMA])
    def k(ts, td, si, di, out_s, out_d, idx_v, rows_v, sem):
        wid = lax.axis_index("s") * NC + lax.axis_index("c")
        base = wid * EW
        for tab, idx, out in ((ts, si, out_s), (td, di, out_d)):
            for h in range(EW // EC):
                off = base + h * EC
                pltpu.sync_copy(idx.at[pl.ds(off, EC)], idx_v)
                pltpu.async_copy(tab.at[idx_v], rows_v, sem).wait()
                pltpu.sync_copy(rows_v, out.at[pl.ds(off, EC)])

    return k(tab_s, tab_d, src, dst)


def _sc_scatter_add(vals, dst, width):
    ECS = 1280
    out_t = jax.ShapeDtypeStruct((NC * N_PAD, width), _f32)

    @functools.partial(
        pl.kernel, out_type=out_t, mesh=_mesh(),
        compiler_params=pltpu.CompilerParams(use_tc_tiling_on_sc=False),
        scratch_types=[pltpu.VMEM((ECS, width), _f32),
                       pltpu.VMEM((ECS,), _i32),
                       pltpu.VMEM_SHARED((N_PAD, width), _f32),
                       pltpu.SemaphoreType.DMA])
    def k(vals_h, dst_h, zr_h, out_h, vals_v, idx_v, accum, sem):
        c = lax.axis_index("c")
        s = lax.axis_index("s")
        wid = s * NC + c
        pltpu.sync_copy(zr_h, vals_v.at[pl.ds(0, NR)])
        pltpu.sync_copy(vals_v.at[pl.ds(0, NR)], accum.at[pl.ds(s * NR, NR)])
        plsc.subcore_barrier()
        for h in range(EW // ECS):
            off = wid * EW + h * ECS
            pltpu.sync_copy(dst_h.at[pl.ds(off, ECS)], idx_v)
            pltpu.sync_copy(vals_h.at[pl.ds(off, ECS)], vals_v)
            pltpu.sync_copy(vals_v, accum.at[idx_v], add=True)
        plsc.subcore_barrier()
        pltpu.sync_copy(accum.at[pl.ds(s * NR, NR)], vals_v.at[pl.ds(0, NR)])
        pltpu.sync_copy(vals_v.at[pl.ds(0, NR)],
                        out_h.at[pl.ds(c * N_PAD + s * NR, NR)])

    zr = jnp.zeros((NR, width), _f32)
    return k(vals, dst, zr).reshape(NC, N_PAD, width)


def _tc_logits1(xs, xd, wlp, wrp, asel):
    def body(xs_r, xd_r, wl_r, wr_r, a_r, lg_r, m_r):
        z = jnp.dot(xs_r[...], wl_r[...], preferred_element_type=_f32)
        z = z + jnp.dot(xd_r[...], wr_r[...], preferred_element_type=_f32)
        z = jnp.where(z > 0, z, 0.2 * z)
        lg = jnp.dot(z, a_r[...], preferred_element_type=_f32)
        lg_r[...] = lg
        bm = jnp.max(lg, axis=0, keepdims=True)
        i = pl.program_id(0)

        @pl.when(i == 0)
        def _():
            m_r[...] = bm

        @pl.when(i > 0)
        def _():
            m_r[...] = jnp.maximum(m_r[...], bm)

    return pl.pallas_call(
        body,
        grid=(E_PAD // BE,),
        in_specs=[pl.BlockSpec((BE, 16), lambda i: (i, 0)),
                  pl.BlockSpec((BE, 16), lambda i: (i, 0)),
                  pl.BlockSpec((16, F1), lambda i: (0, 0)),
                  pl.BlockSpec((16, F1), lambda i: (0, 0)),
                  pl.BlockSpec((F1, 8), lambda i: (0, 0))],
        out_specs=[pl.BlockSpec((BE, 8), lambda i: (i, 0)),
                   pl.BlockSpec((1, 8), lambda i: (0, 0))],
        out_shape=[jax.ShapeDtypeStruct((E_PAD, 8), _f32),
                   jax.ShapeDtypeStruct((1, 8), _f32)],
    )(xs, xd, wlp, wrp, asel)


def _tc_scat1(lg, m, xs, rm, tm, bm):
    def body(lg_r, m_r, xs_r, r_r, t_r, b_r, o_r):
        i = pl.program_id(0)
        ex = jnp.exp(lg_r[...] - m_r[...])
        left = jnp.dot(ex, r_r[...], preferred_element_type=_f32)
        right = jnp.dot(xs_r[...], t_r[...], preferred_element_type=_f32) + b_r[...]
        rows = i * BE + lax.broadcasted_iota(_i32, (BE, 1), 0)
        o_r[...] = left * right * (rows < E).astype(_f32)

    return pl.pallas_call(
        body,
        grid=(E_PAD // BE,),
        in_specs=[pl.BlockSpec((BE, 8), lambda i: (i, 0)),
                  pl.BlockSpec((1, 8), lambda i: (0, 0)),
                  pl.BlockSpec((BE, 16), lambda i: (i, 0)),
                  pl.BlockSpec((8, 48), lambda i: (0, 0)),
                  pl.BlockSpec((16, 48), lambda i: (0, 0)),
                  pl.BlockSpec((1, 48), lambda i: (0, 0))],
        out_specs=pl.BlockSpec((BE, 48), lambda i: (i, 0)),
        out_shape=jax.ShapeDtypeStruct((E_PAD, 48), _f32),
    )(lg, m, xs, rm, tm, bm)


def _tc_finish1(acc, keep, w1big48, s48, b1r, wl2p, wr2p):
    def body(a_r, k_r, w_r, s_r, b_r, wl_r, wr_r, xl_r, xr_r):
        a = a_r[0] + a_r[1]
        z = jnp.dot(a, w_r[...], preferred_element_type=_f32)
        de = jnp.dot(a, s_r[...], preferred_element_type=_f32)
        h = z / (de + 1e-16) + b_r[...]
        h = jnp.maximum(h, 0.0)
        h = h * k_r[...].astype(_f32)
        xl_r[...] = jnp.dot(h, wl_r[...], preferred_element_type=_f32)
        xr_r[...] = jnp.dot(h, wr_r[...], preferred_element_type=_f32)

    return pl.pallas_call(
        body,
        grid=(N // BN,),
        in_specs=[pl.BlockSpec((2, BN, 48), lambda i: (0, i, 0)),
                  pl.BlockSpec((BN, F1), lambda i: (i, 0)),
                  pl.BlockSpec((48, F1), lambda i: (0, 0)),
                  pl.BlockSpec((48, F1), lambda i: (0, 0)),
                  pl.BlockSpec((1, F1), lambda i: (0, 0)),
                  pl.BlockSpec((F1, 16), lambda i: (0, 0)),
                  pl.BlockSpec((F1, 16), lambda i: (0, 0))],
        out_specs=[pl.BlockSpec((BN, 16), lambda i: (i, 0)),
                   pl.BlockSpec((BN, 16), lambda i: (i, 0))],
        out_shape=[jax.ShapeDtypeStruct((N, 16), _f32),
                   jax.ShapeDtypeStruct((N, 16), _f32)],
    )(acc, keep, w1big48, s48, b1r, wl2p, wr2p)


def _tc_logits2(xls, xrd, att2row):
    def body(l_r, r_r, a_r, lg_r, m_r):
        z = l_r[...] + r_r[...]
        z = jnp.where(z > 0, z, 0.2 * z)
        lg = jnp.sum(z * a_r[...], axis=1, keepdims=True)
        lg_r[...] = lg
        bm = jnp.max(lg, axis=0, keepdims=True)
        i = pl.program_id(0)

        @pl.when(i == 0)
        def _():
            m_r[...] = bm

        @pl.when(i > 0)
        def _():
            m_r[...] = jnp.maximum(m_r[...], bm)

    return pl.pallas_call(
        body,
        grid=(E_PAD // BE,),
        in_specs=[pl.BlockSpec((BE, 16), lambda i: (i, 0)),
                  pl.BlockSpec((BE, 16), lambda i: (i, 0)),
                  pl.BlockSpec((1, 16), lambda i: (0, 0))],
        out_specs=[pl.BlockSpec((BE, 1), lambda i: (i, 0)),
                   pl.BlockSpec((1, 1), lambda i: (0, 0))],
        out_shape=[jax.ShapeDtypeStruct((E_PAD, 1), _f32),
                   jax.ShapeDtypeStruct((1, 1), _f32)],
    )(xls, xrd, att2row)


def _tc_scat2(lg2, m2, xls, c16, b16):
    def body(lg_r, m_r, x_r, c_r, b_r, o_r):
        i = pl.program_id(0)
        ex = jnp.exp(lg_r[...] - m_r[...])
        t = jnp.dot(x_r[...], c_r[...], preferred_element_type=_f32) + b_r[...]
        rows = i * BE + lax.broadcasted_iota(_i32, (BE, 1), 0)
        o_r[...] = ex * t * (rows < E).astype(_f32)

    return pl.pallas_call(
        body,
        grid=(E_PAD // BE,),
        in_specs=[pl.BlockSpec((BE, 1), lambda i: (i, 0)),
                  pl.BlockSpec((1, 1), lambda i: (0, 0)),
                  pl.BlockSpec((BE, 16), lambda i: (i, 0)),
                  pl.BlockSpec((16, 16), lambda i: (0, 0)),
                  pl.BlockSpec((1, 16), lambda i: (0, 0))],
        out_specs=pl.BlockSpec((BE, 16), lambda i: (i, 0)),
        out_shape=jax.ShapeDtypeStruct((E_PAD, 16), _f32),
    )(lg2, m2, xls, c16, b16)


def _tc_final(acc2, b2r):
    def body(a_r, b_r, o_r):
        s = a_r[0] + a_r[1]
        o_r[...] = s[:, 1:5] / (s[:, 0:1] + 1e-16) + b_r[...]

    return pl.pallas_call(
        body,
        grid=(1,),
        in_specs=[pl.BlockSpec((2, N, 16), lambda i: (0, 0, 0)),
                  pl.BlockSpec((1, 4), lambda i: (0, 0))],
        out_specs=pl.BlockSpec((N, 4), lambda i: (0, 0)),
        out_shape=jax.ShapeDtypeStruct((N, 4), _f32),
    )(acc2, b2r)


def _np_consts():
    hm = (np.arange(F1)[:, None] // D1 == np.arange(H1)[None, :])
    hm32 = np.zeros((32, F1), np.float32)
    for h in range(H1):
        hm32[h * 4:(h + 1) * 4, h * D1:(h + 1) * D1] = 1.0
    s48 = np.zeros((48, F1), np.float32)
    for h in range(H1):
        s48[h, h * D1:(h + 1) * D1] = 1.0
    r48 = np.zeros((8, 48), np.float32)
    t48 = np.zeros((16, 48), np.float32)
    b48 = np.zeros((1, 48), np.float32)
    for h in range(H1):
        r48[h, h] = 1.0
        b48[0, h] = 1.0
        for kk in range(4):
            r48[h, 8 + h * 4 + kk] = 1.0
            t48[kk, 8 + h * 4 + kk] = 1.0
    c16 = np.zeros((16, 16), np.float32)
    b16 = np.zeros((1, 16), np.float32)
    b16[0, 0] = 1.0
    for kk in range(4):
        c16[kk, 1 + kk] = 1.0
    return (hm.astype(np.float32), hm32, s48, r48, t48, b48, c16, b16)


(_HM, _HM32, _S48, _R48, _T48, _B48, _C16, _B16) = _np_consts()

_KEEP_CACHE = None


def _np_threefry2x32(k0, k1, x0, x1):
    def rotl(x, d):
        return ((x << np.uint32(d)) | (x >> np.uint32(32 - d))).astype(np.uint32)
    ks = [np.uint32(k0), np.uint32(k1),
          np.uint32(k0) ^ np.uint32(k1) ^ np.uint32(0x1BD11BDA)]
    rotations = [(13, 15, 26, 6), (17, 29, 16, 24)]
    x = [x0.astype(np.uint32) + ks[0], x1.astype(np.uint32) + ks[1]]
    for i in range(5):
        for r in rotations[i % 2]:
            x[0] = (x[0] + x[1]).astype(np.uint32)
            x[1] = rotl(x[1], r) ^ x[0]
        x[0] = (x[0] + ks[(i + 1) % 3]).astype(np.uint32)
        x[1] = (x[1] + ks[(i + 2) % 3] + np.uint32(i + 1)).astype(np.uint32)
    return x[0], x[1]


def _keep_const():
    global _KEEP_CACHE
    if _KEEP_CACHE is None:
        size = N * F1
        i = np.arange(size, dtype=np.uint64)
        x0 = (i >> np.uint64(32)).astype(np.uint32)
        x1 = (i & np.uint64(0xFFFFFFFF)).astype(np.uint32)
        o0, o1 = _np_threefry2x32(0, 123, x0, x1)
        bits = (o0 ^ o1).reshape(N, F1)
        _KEEP_CACHE = (bits < np.uint32(1 << 31)).astype(np.int8) * np.int8(2)
    return _KEEP_CACHE


def kernel(x, edge_index, Wl1, Wr1, att1, b1, Wl2, Wr2, att2, b2):
    src = edge_index[0].astype(_i32)
    dst = edge_index[1].astype(_i32)
    pad = jnp.zeros((E_PAD - E,), _i32)
    src_p = jnp.concatenate([src, pad])
    dst_p = jnp.concatenate([dst, pad])
    xpad = jnp.concatenate([x, jnp.zeros((N, 16 - IN), _f32)], axis=1)

    wlp = jnp.concatenate([Wl1, jnp.zeros((12, F1), _f32)], axis=0)
    wrp = jnp.concatenate([Wr1, jnp.zeros((12, F1), _f32)], axis=0)
    asel = att1.reshape(F1)[:, None] * _HM
    w1big48 = jnp.tile(Wl1, (H1, 1)) * _HM32
    w1big48 = jnp.concatenate(
        [jnp.zeros((8, F1), _f32), w1big48, jnp.zeros((8, F1), _f32)], axis=0)
    wl2p = jnp.concatenate([Wl2, jnp.zeros((F1, 12), _f32)], axis=1)
    wr2p = jnp.concatenate([Wr2, jnp.zeros((F1, 12), _f32)], axis=1)
    att2row = jnp.concatenate(
        [att2.reshape(1, D2), jnp.zeros((1, 12), _f32)], axis=1)
    b1r = b1.reshape(1, F1)
    b2r = b2.reshape(1, D2)
    keep = jnp.asarray(_keep_const())

    xs, xd = _sc_gather(xpad, xpad, src_p, dst_p)
    lg1, m1 = _tc_logits1(xs, xd, wlp, wrp, asel)
    scat1 = _tc_scat1(lg1, m1, xs, _R48, _T48, _B48)
    acc1 = _sc_scatter_add(scat1, dst_p, 48)[:, :N, :]
    xl2p, xr2p = _tc_finish1(acc1, keep, w1big48, _S48, b1r, wl2p, wr2p)

    xls, xrd = _sc_gather(xl2p, xr2p, src_p, dst_p)
    lg2, m2 = _tc_logits2(xls, xrd, att2row)
    scat2 = _tc_scat2(lg2, m2, xls, _C16, _B16)
    acc2 = _sc_scatter_add(scat2, dst_p, 16)[:, :N, :]
    return _tc_final(acc2, b2r)


# speedup vs baseline: 27.1419x; 1.3678x over previous
import functools

import numpy as np
import jax
import jax.numpy as jnp
from jax import lax
from jax.experimental import pallas as pl
from jax.experimental.pallas import tpu as pltpu
from jax.experimental.pallas import tpu_sc as plsc

_f32 = jnp.float32
_i32 = jnp.int32

N = 10000
E = 160000
IN = 4
H1, D1 = 8, 120
F1 = H1 * D1
D2 = 4

NC, NS = 2, 16
NW = NC * NS
E_PAD = 163840
EW = E_PAD // NW
EC = 2560
N_PAD = 10240
NR = N_PAD // NS

BE = 2560
BN = 2000
G8 = BE // 8
G16 = BE // 16


def _mesh():
    return plsc.VectorSubcoreMesh(
        core_axis_name="c", subcore_axis_name="s",
        num_cores=NC, num_subcores=NS)


def _unpack8(p):
    return jnp.concatenate([p[:, 16 * j:16 * (j + 1)] for j in range(8)], axis=0)


def _pack8(x):
    return jnp.concatenate([x[G8 * j:G8 * (j + 1), :] for j in range(8)], axis=1)


def _unpack16(p):
    return jnp.concatenate([p[:, 8 * j:8 * (j + 1)] for j in range(16)], axis=0)


def _pack16(x):
    return jnp.concatenate([x[G16 * j:G16 * (j + 1), :] for j in range(16)], axis=1)


def _packed_edge_mask(i):
    rows = lax.broadcasted_iota(_i32, (G8, 128), 0) * 8
    lanes = lax.broadcasted_iota(_i32, (G8, 128), 1) // 16
    return ((i * BE + rows + lanes) < E).astype(_f32)


def _sc_gather(tab_s, tab_d, src, dst):
    out_t = (jax.ShapeDtypeStruct((E_PAD, 16), _f32),
             jax.ShapeDtypeStruct((E_PAD, 16), _f32))

    @functools.partial(
        pl.kernel, out_type=out_t, mesh=_mesh(),
        compiler_params=pltpu.CompilerParams(use_tc_tiling_on_sc=False),
        scratch_types=[pltpu.VMEM((EC,), _i32),
                       pltpu.VMEM((EC, 16), _f32),
                       pltpu.SemaphoreType.DMA])
    def k(ts, td, si, di, out_s, out_d, idx_v, rows_v, sem):
        wid = lax.axis_index("s") * NC + lax.axis_index("c")
        base = wid * EW
        for tab, idx, out in ((ts, si, out_s), (td, di, out_d)):
            for h in range(EW // EC):
                off = base + h * EC
                pltpu.sync_copy(idx.at[pl.ds(off, EC)], idx_v)
                pltpu.async_copy(tab.at[idx_v], rows_v, sem).wait()
                pltpu.sync_copy(rows_v, out.at[pl.ds(off, EC)])

    xs, xd = k(tab_s, tab_d, src, dst)
    return xs.reshape(E_PAD // 8, 128), xd.reshape(E_PAD // 8, 128)


def _sc_scatter_add(vals_list, dst):
    P = len(vals_list)
    ECS = 1280
    out_t = jax.ShapeDtypeStruct((P, NC * N_PAD, 16), _f32)
    scratch = ([pltpu.VMEM((ECS, 16), _f32) for _ in range(P)]
               + [pltpu.VMEM((ECS,), _i32)]
               + [pltpu.VMEM_SHARED((N_PAD, 16), _f32) for _ in range(P)]
               + [pltpu.SemaphoreType.DMA])

    @functools.partial(
        pl.kernel, out_type=out_t, mesh=_mesh(),
        compiler_params=pltpu.CompilerParams(use_tc_tiling_on_sc=False),
        scratch_types=scratch)
    def k(*refs):
        vals_h = refs[:P]
        dst_h, zr_h, out_h = refs[P], refs[P + 1], refs[P + 2]
        vals_v = refs[P + 3:2 * P + 3]
        idx_v = refs[2 * P + 3]
        accum = refs[2 * P + 4:3 * P + 4]
        c = lax.axis_index("c")
        s = lax.axis_index("s")
        wid = s * NC + c
        pltpu.sync_copy(zr_h, vals_v[0].at[pl.ds(0, NR)])
        for p in range(P):
            pltpu.sync_copy(vals_v[0].at[pl.ds(0, NR)],
                            accum[p].at[pl.ds(s * NR, NR)])
        plsc.subcore_barrier()
        for h in range(EW // ECS):
            off = wid * EW + h * ECS
            pltpu.sync_copy(dst_h.at[pl.ds(off, ECS)], idx_v)
            for p in range(P):
                pltpu.sync_copy(vals_h[p].at[pl.ds(off, ECS)], vals_v[p])
            for p in range(P):
                pltpu.sync_copy(vals_v[p], accum[p].at[idx_v], add=True)
        plsc.subcore_barrier()
        for p in range(P):
            pltpu.sync_copy(accum[p].at[pl.ds(s * NR, NR)],
                            vals_v[p].at[pl.ds(0, NR)])
            pltpu.sync_copy(vals_v[p].at[pl.ds(0, NR)],
                            out_h.at[p].at[pl.ds(c * N_PAD + s * NR, NR)])

    zr = jnp.zeros((NR, 16), _f32)
    flat = [v.reshape(E_PAD, 16) for v in vals_list]
    return k(*flat, dst, zr).reshape(P, NC, N_PAD, 16)


def _tc_logits1(xs, xd, wlp, wrp, asel):
    def body(xs_r, xd_r, wl_r, wr_r, a_r, lg_r, m_r):
        z = jnp.dot(_unpack8(xs_r[...]), wl_r[...], preferred_element_type=_f32)
        z = z + jnp.dot(_unpack8(xd_r[...]), wr_r[...],
                        preferred_element_type=_f32)
        z = jnp.where(z > 0, z, 0.2 * z)
        lg = jnp.dot(z, a_r[...], preferred_element_type=_f32)
        lg_r[...] = _pack16(lg)
        bm = jnp.max(lg, axis=0, keepdims=True)
        i = pl.program_id(0)

        @pl.when(i == 0)
        def _():
            m_r[...] = bm

        @pl.when(i > 0)
        def _():
            m_r[...] = jnp.maximum(m_r[...], bm)

    return pl.pallas_call(
        body,
        grid=(E_PAD // BE,),
        in_specs=[pl.BlockSpec((G8, 128), lambda i: (i, 0)),
                  pl.BlockSpec((G8, 128), lambda i: (i, 0)),
                  pl.BlockSpec((16, F1), lambda i: (0, 0)),
                  pl.BlockSpec((16, F1), lambda i: (0, 0)),
                  pl.BlockSpec((F1, 8), lambda i: (0, 0))],
        out_specs=[pl.BlockSpec((G16, 128), lambda i: (i, 0)),
                   pl.BlockSpec((1, 8), lambda i: (0, 0))],
        out_shape=[jax.ShapeDtypeStruct((E_PAD // 16, 128), _f32),
                   jax.ShapeDtypeStruct((1, 8), _f32)],
    )(xs, xd, wlp, wrp, asel)


def _tc_scat1(lg, m, xs, sels):
    def body(lg_r, m_r, xs_r, r0, t0, b0, r1, t1, b1, r2, t2, b2,
             o0, o1, o2):
        i = pl.program_id(0)
        ex = jnp.exp(_unpack16(lg_r[...]) - m_r[...])
        xsv = _unpack8(xs_r[...])
        msk = _packed_edge_mask(i)
        for r_r, t_r, b_r, o_r in ((r0, t0, b0, o0), (r1, t1, b1, o1),
                                   (r2, t2, b2, o2)):
            left = jnp.dot(ex, r_r[...], preferred_element_type=_f32)
            right = jnp.dot(xsv, t_r[...],
                            preferred_element_type=_f32) + b_r[...]
            o_r[...] = _pack8(left * right) * msk

    sel_specs = []
    for _ in range(3):
        sel_specs += [pl.BlockSpec((8, 16), lambda i: (0, 0)),
                      pl.BlockSpec((16, 16), lambda i: (0, 0)),
                      pl.BlockSpec((1, 16), lambda i: (0, 0))]
    return pl.pallas_call(
        body,
        grid=(E_PAD // BE,),
        in_specs=[pl.BlockSpec((G16, 128), lambda i: (i, 0)),
                  pl.BlockSpec((1, 8), lambda i: (0, 0)),
                  pl.BlockSpec((G8, 128), lambda i: (i, 0))] + sel_specs,
        out_specs=[pl.BlockSpec((G8, 128), lambda i: (i, 0))] * 3,
        out_shape=[jax.ShapeDtypeStruct((E_PAD // 8, 128), _f32)] * 3,
    )(lg, m, xs, *sels)


def _tc_finish1(acc, keep, w1big48, s48, b1r, wl2p, wr2p):
    def body(a_r, k_r, w_r, s_r, b_r, wl_r, wr_r, xl_r, xr_r):
        a = jnp.concatenate(
            [a_r[p, 0] + a_r[p, 1] for p in range(3)], axis=1)
        z = jnp.dot(a, w_r[...], preferred_element_type=_f32)
        de = jnp.dot(a, s_r[...], preferred_element_type=_f32)
        h = z / (de + 1e-16) + b_r[...]
        h = jnp.maximum(h, 0.0)
        h = h * k_r[...].astype(_f32)
        xl_r[...] = jnp.dot(h, wl_r[...], preferred_element_type=_f32)
        xr_r[...] = jnp.dot(h, wr_r[...], preferred_element_type=_f32)

    return pl.pallas_call(
        body,
        grid=(N // BN,),
        in_specs=[pl.BlockSpec((3, 2, BN, 16), lambda i: (0, 0, i, 0)),
                  pl.BlockSpec((BN, F1), lambda i: (i, 0)),
                  pl.BlockSpec((48, F1), lambda i: (0, 0)),
                  pl.BlockSpec((48, F1), lambda i: (0, 0)),
                  pl.BlockSpec((1, F1), lambda i: (0, 0)),
                  pl.BlockSpec((F1, 16), lambda i: (0, 0)),
                  pl.BlockSpec((F1, 16), lambda i: (0, 0))],
        out_specs=[pl.BlockSpec((BN, 16), lambda i: (i, 0)),
                   pl.BlockSpec((BN, 16), lambda i: (i, 0))],
        out_shape=[jax.ShapeDtypeStruct((N, 16), _f32),
                   jax.ShapeDtypeStruct((N, 16), _f32)],
    )(acc, keep, w1big48, s48, b1r, wl2p, wr2p)


def _tc_logits2(xls, xrd, att2c):
    def body(l_r, r_r, a_r, lg_r, m_r):
        z = _unpack8(l_r[...]) + _unpack8(r_r[...])
        z = jnp.where(z > 0, z, 0.2 * z)
        lg = jnp.dot(z, a_r[...], preferred_element_type=_f32)
        lg_r[...] = _pack16(lg)
        bm = jnp.max(lg, axis=0, keepdims=True)
        i = pl.program_id(0)

        @pl.when(i == 0)
        def _():
            m_r[...] = bm

        @pl.when(i > 0)
        def _():
            m_r[...] = jnp.maximum(m_r[...], bm)

    return pl.pallas_call(
        body,
        grid=(E_PAD // BE,),
        in_specs=[pl.BlockSpec((G8, 128), lambda i: (i, 0)),
                  pl.BlockSpec((G8, 128), lambda i: (i, 0)),
                  pl.BlockSpec((16, 8), lambda i: (0, 0))],
        out_specs=[pl.BlockSpec((G16, 128), lambda i: (i, 0)),
                   pl.BlockSpec((1, 8), lambda i: (0, 0))],
        out_shape=[jax.ShapeDtypeStruct((E_PAD // 16, 128), _f32),
                   jax.ShapeDtypeStruct((1, 8), _f32)],
    )(xls, xrd, att2c)


def _tc_scat2(lg2, m2, xls, c16, b16):
    def body(lg_r, m_r, x_r, c_r, b_r, o_r):
        i = pl.program_id(0)
        ex = jnp.exp(_unpack16(lg_r[...])[:, 0:1] - m_r[0, 0])
        t = jnp.dot(_unpack8(x_r[...]), c_r[...],
                    preferred_element_type=_f32) + b_r[...]
        o_r[...] = _pack8(ex * t) * _packed_edge_mask(i)

    return pl.pallas_call(
        body,
        grid=(E_PAD // BE,),
        in_specs=[pl.BlockSpec((G16, 128), lambda i: (i, 0)),
                  pl.BlockSpec((1, 8), lambda i: (0, 0)),
                  pl.BlockSpec((G8, 128), lambda i: (i, 0)),
                  pl.BlockSpec((16, 16), lambda i: (0, 0)),
                  pl.BlockSpec((1, 16), lambda i: (0, 0))],
        out_specs=pl.BlockSpec((G8, 128), lambda i: (i, 0)),
        out_shape=jax.ShapeDtypeStruct((E_PAD // 8, 128), _f32),
    )(lg2, m2, xls, c16, b16)


def _tc_final(acc2, b2r):
    def body(a_r, b_r, o_r):
        s = a_r[0, 0] + a_r[0, 1]
        o_r[...] = s[:, 1:5] / (s[:, 0:1] + 1e-16) + b_r[...]

    return pl.pallas_call(
        body,
        grid=(1,),
        in_specs=[pl.BlockSpec((1, 2, N, 16), lambda i: (0, 0, 0, 0)),
                  pl.BlockSpec((1, 4), lambda i: (0, 0))],
        out_specs=pl.BlockSpec((N, 4), lambda i: (0, 0)),
        out_shape=jax.ShapeDtypeStruct((N, 4), _f32),
    )(acc2, b2r)


def _np_consts():
    hm = (np.arange(F1)[:, None] // D1 == np.arange(H1)[None, :])
    hm32 = np.zeros((32, F1), np.float32)
    for h in range(H1):
        hm32[h * 4:(h + 1) * 4, h * D1:(h + 1) * D1] = 1.0
    s48 = np.zeros((48, F1), np.float32)
    for h in range(H1):
        s48[h, h * D1:(h + 1) * D1] = 1.0
    sels = []
    for p in range(3):
        r = np.zeros((8, 16), np.float32)
        t = np.zeros((16, 16), np.float32)
        b = np.zeros((1, 16), np.float32)
        for h in range(H1):
            c = h
            if p == 0 and c < 16:
                r[h, c] = 1.0
                b[0, c] = 1.0
            for kk in range(4):
                c = 8 + 4 * h + kk
                if 16 * p <= c < 16 * (p + 1):
                    r[h, c - 16 * p] = 1.0
                    t[kk, c - 16 * p] = 1.0
        sels.append((r, t, b))
    c16 = np.zeros((16, 16), np.float32)
    b16 = np.zeros((1, 16), np.float32)
    b16[0, 0] = 1.0
    for kk in range(4):
        c16[kk, 1 + kk] = 1.0
    return hm.astype(np.float32), hm32, s48, sels, c16, b16


(_HM, _HM32, _S48, _SELS, _C16, _B16) = _np_consts()

_KEEP_CACHE = None


def _np_threefry2x32(k0, k1, x0, x1):
    def rotl(x, d):
        return ((x << np.uint32(d)) | (x >> np.uint32(32 - d))).astype(np.uint32)
    ks = [np.uint32(k0), np.uint32(k1),
          np.uint32(k0) ^ np.uint32(k1) ^ np.uint32(0x1BD11BDA)]
    rotations = [(13, 15, 26, 6), (17, 29, 16, 24)]
    x = [x0.astype(np.uint32) + ks[0], x1.astype(np.uint32) + ks[1]]
    for i in range(5):
        for r in rotations[i % 2]:
            x[0] = (x[0] + x[1]).astype(np.uint32)
            x[1] = rotl(x[1], r) ^ x[0]
        x[0] = (x[0] + ks[(i + 1) % 3]).astype(np.uint32)
        x[1] = (x[1] + ks[(i + 2) % 3] + np.uint32(i + 1)).astype(np.uint32)
    return x[0], x[1]


def _keep_const():
    global _KEEP_CACHE
    if _KEEP_CACHE is None:
        size = N * F1
        i = np.arange(size, dtype=np.uint64)
        x0 = (i >> np.uint64(32)).astype(np.uint32)
        x1 = (i & np.uint64(0xFFFFFFFF)).astype(np.uint32)
        o0, o1 = _np_threefry2x32(0, 123, x0, x1)
        bits = (o0 ^ o1).reshape(N, F1)
        _KEEP_CACHE = (bits < np.uint32(1 << 31)).astype(np.int8) * np.int8(2)
    return _KEEP_CACHE


def kernel(x, edge_index, Wl1, Wr1, att1, b1, Wl2, Wr2, att2, b2):
    src = edge_index[0].astype(_i32)
    dst = edge_index[1].astype(_i32)
    pad = jnp.zeros((E_PAD - E,), _i32)
    src_p = jnp.concatenate([src, pad])
    dst_p = jnp.concatenate([dst, pad])
    xpad = jnp.concatenate([x, jnp.zeros((N, 16 - IN), _f32)], axis=1)

    wlp = jnp.concatenate([Wl1, jnp.zeros((12, F1), _f32)], axis=0)
    wrp = jnp.concatenate([Wr1, jnp.zeros((12, F1), _f32)], axis=0)
    asel = att1.reshape(F1)[:, None] * _HM
    w1big48 = jnp.tile(Wl1, (H1, 1)) * _HM32
    w1big48 = jnp.concatenate(
        [jnp.zeros((8, F1), _f32), w1big48, jnp.zeros((8, F1), _f32)], axis=0)
    wl2p = jnp.concatenate([Wl2, jnp.zeros((F1, 12), _f32)], axis=1)
    wr2p = jnp.concatenate([Wr2, jnp.zeros((F1, 12), _f32)], axis=1)
    att2c = jnp.concatenate(
        [att2.reshape(D2, 1), jnp.zeros((D2, 7), _f32)], axis=1)
    att2c = jnp.concatenate([att2c, jnp.zeros((12, 8), _f32)], axis=0)
    b1r = b1.reshape(1, F1)
    b2r = b2.reshape(1, D2)
    keep = jnp.asarray(_keep_const())
    sels = [jnp.asarray(s) for rtb in _SELS for s in rtb]

    xs, xd = _sc_gather(xpad, xpad, src_p, dst_p)
    lg1, m1 = _tc_logits1(xs, xd, wlp, wrp, asel)
    s1a, s1b, s1c = _tc_scat1(lg1, m1, xs, sels)
    acc1 = _sc_scatter_add([s1a, s1b, s1c], dst_p)
    xl2p, xr2p = _tc_finish1(acc1, keep, w1big48, _S48, b1r, wl2p, wr2p)

    xls, xrd = _sc_gather(xl2p, xr2p, src_p, dst_p)
    lg2, m2 = _tc_logits2(xls, xrd, att2c)
    scat2 = _tc_scat2(lg2, m2, xls, _C16, _B16)
    acc2 = _sc_scatter_add([scat2], dst_p)
    return _tc_final(acc2, b2r)
